# baseline (speedup 1.0000x reference)
import functools

import jax
import jax.numpy as jnp
from jax import lax
from jax.experimental import pallas as pl
from jax.experimental.pallas import tpu as pltpu
from jax.experimental.pallas import tpu_sc as plsc

_S = 2048
_H = 16
_PAD = 4096
_NSHIFT = 8
_K = 8


def _expand_bias(shifted_tables):
    mesh = plsc.VectorSubcoreMesh(core_axis_name="c", subcore_axis_name="s")

    @functools.partial(
        pl.kernel,
        mesh=mesh,
        out_type=jax.ShapeDtypeStruct((1, _H, _S, _S), jnp.float32),
        scratch_types=[
            pltpu.VMEM((_NSHIFT * _PAD,), jnp.float32),
            pltpu.SemaphoreType.DMA,
        ],
        compiler_params=pltpu.CompilerParams(use_tc_tiling_on_sc=False),
    )
    def body(p_hbm, out_hbm, p_v, sem):
        cid = lax.axis_index("c")
        sid = lax.axis_index("s")
        wid = sid * 2 + cid
        h = wid // 2
        i0 = (wid % 2) * (_S // 2)

        pltpu.sync_copy(p_hbm.at[h], p_v)

        def row_copy(i, kk):
            off = (_S - 1) - i
            q = ((_S - 1) - kk) % _NSHIFT
            base = q * _PAD + (off - q)
            return pltpu.make_async_copy(
                p_v.at[pl.ds(base, _S)],
                out_hbm.at[0, h, i],
                sem,
            )

        def fire(g):
            ibase = i0 + g * _K
            for kk in range(_K):
                row_copy(ibase + kk, kk).start()

        def drain(g):
            ibase = i0 + g * _K
            for kk in range(_K):
                row_copy(ibase + kk, kk).wait()

        ngroups = (_S // 2) // _K
        fire(0)
        fire(1)

        def loop(g, carry):
            drain(g)
            fire(g + 2)
            return carry

        lax.fori_loop(0, ngroups - 2, loop, 0)
        drain(ngroups - 2)
        drain(ngroups - 1)

    return body(shifted_tables)


def kernel(seq_len, bias_table):
    del seq_len
    rev = bias_table[::-1, :]
    pad_rows = _PAD + _NSHIFT - 1 - rev.shape[0]
    rev = jnp.concatenate(
        [rev, jnp.zeros((pad_rows, _H), bias_table.dtype)], axis=0)
    shifted = jnp.stack(
        [lax.slice_in_dim(rev, q, q + _PAD, axis=0) for q in range(_NSHIFT)],
        axis=0)
    shifted = jnp.transpose(shifted, (2, 0, 1)).reshape(_H, _NSHIFT * _PAD)
    return _expand_bias(shifted)


# speedup vs baseline: 1.6636x; 1.6636x over previous
import functools

import jax
import jax.numpy as jnp
from jax import lax
from jax.experimental import pallas as pl
from jax.experimental.pallas import tpu as pltpu
from jax.experimental.pallas import tpu_sc as plsc

_S = 2048
_H = 16
_PAD = 4096
_BLK = 8
_W = 2064


def _expand_bias(rev_flat):
    mesh = plsc.VectorSubcoreMesh(core_axis_name="c", subcore_axis_name="s")
    blocks_per_worker = (_S // _BLK) // 2
    npairs = blocks_per_worker // 2

    @functools.partial(
        pl.kernel,
        mesh=mesh,
        out_type=jax.ShapeDtypeStruct((1, _H, _S, _S), jnp.float32),
        scratch_types=[
            pltpu.VMEM((_W,), jnp.float32),
            pltpu.VMEM((_W,), jnp.float32),
            pltpu.VMEM((_BLK, _S), jnp.float32),
            pltpu.VMEM((_BLK, _S), jnp.float32),
            pltpu.SemaphoreType.DMA,
            pltpu.SemaphoreType.DMA,
            pltpu.SemaphoreType.DMA,
            pltpu.SemaphoreType.DMA,
        ],
    )
    def body(tab_hbm, out_hbm, win0, win1, stage0, stage1,
             wsem0, wsem1, osem0, osem1):
        cid = lax.axis_index("c")
        sid = lax.axis_index("s")
        wid = sid * 2 + cid
        h = wid // 2
        t0 = (wid % 2) * blocks_per_worker

        def win_copy(win, wsem, ti, lead):
            start = pl.multiple_of(
                h * _PAD + (_S - 1) - ti * _BLK - lead, 16)
            return pltpu.make_async_copy(
                tab_hbm.at[pl.ds(start, _W)], win, wsem)

        def out_copy(stage, osem, ti):
            return pltpu.make_async_copy(
                stage,
                out_hbm.at[0, h, pl.ds(ti * _BLK, _BLK), :],
                osem,
            )

        def build(stage, win, lead):
            for k in range(_S // 128):
                for r in range(_BLK):
                    base = lead - r + k * 128
                    for c in range(8):
                        stage[r, pl.ds(k * 128 + c * 16, 16)] = (
                            win[pl.ds(base + c * 16, 16)]
                        )

        win_copy(win0, wsem0, t0, 15).start()
        win_copy(win1, wsem1, t0 + 1, 7).start()

        def loop(g, carry):
            ti = t0 + 2 * g
            win_copy(win0, wsem0, ti, 15).wait()

            @pl.when(g > 0)
            def _():
                out_copy(stage0, osem0, ti - 2).wait()

            build(stage0, win0, 15)
            out_copy(stage0, osem0, ti).start()

            @pl.when(g < npairs - 1)
            def _():
                win_copy(win0, wsem0, ti + 2, 15).start()

            win_copy(win1, wsem1, ti + 1, 7).wait()

            @pl.when(g > 0)
            def _():
                out_copy(stage1, osem1, ti - 1).wait()

            build(stage1, win1, 7)
            out_copy(stage1, osem1, ti + 1).start()

            @pl.when(g < npairs - 1)
            def _():
                win_copy(win1, wsem1, ti + 3, 7).start()

            return carry

        lax.fori_loop(0, npairs, loop, 0)
        out_copy(stage0, osem0, t0 + blocks_per_worker - 2).wait()
        out_copy(stage1, osem1, t0 + blocks_per_worker - 1).wait()

    return body(rev_flat)


def kernel(seq_len, bias_table):
    del seq_len
    rev = bias_table[::-1, :]
    rev = jnp.concatenate(
        [rev, jnp.zeros((_PAD - rev.shape[0], _H), bias_table.dtype)], axis=0)
    rev_flat = jnp.transpose(rev, (1, 0)).reshape(_H * _PAD)
    return _expand_bias(rev_flat)
